# baseline (speedup 1.0000x reference)
import functools

import jax
import jax.numpy as jnp
from jax import lax
from jax.experimental import pallas as pl
from jax.experimental.pallas import tpu as pltpu
from jax.experimental.pallas import tpu_sc as plsc

_LANES = 16
_CH = 128
_NC = 2
_NS = 16
_NW = _NC * _NS
_NSEG_PAD = 1024
_PTR_BUF = _NSEG_PAD + 2 * _LANES
_FB = 32


def _tc_mlp_body(w1x_ref, w1h_ref, w2_ref, b1_ref, b2_ref, x_ref, h_ref,
                 out_ref):
    xb = x_ref[...].astype(jnp.bfloat16)
    hb = h_ref[...].astype(jnp.bfloat16)
    hidden = jnp.maximum(
        jnp.dot(xb, w1x_ref[...], preferred_element_type=jnp.float32)
        + jnp.dot(hb, w1h_ref[...], preferred_element_type=jnp.float32)
        + b1_ref[...], 0.0)
    out_ref[...] = jnp.dot(hidden.astype(jnp.bfloat16), w2_ref[...],
                           preferred_element_type=jnp.float32) + b2_ref[...]


def _select_body(end_ref, a_ref, b_ref, out_ref, *, r_split):
    out_ref[...] = jnp.where(end_ref[...] < r_split, a_ref[...], b_ref[...])


def _extract(vec_ref, j):
    v = vec_ref[pl.ds(j, _LANES)]
    return v[0]


def _count_below(ptr_v, thresh):
    base = jnp.int32(0)
    step = _NSEG_PAD
    while step >= 1:
        nxt = base + step
        safe = jnp.minimum(nxt, _PTR_BUF - _LANES)
        v = _extract(ptr_v, safe)
        take = jnp.logical_and(v < thresh, nxt <= _NSEG_PAD)
        base = jnp.where(take, nxt, base)
        step //= 2
    return base


def _make_sc_body(row_lo, row_hi, is_last):
    rw = (row_hi - row_lo) // _NW

    def body(h_hbm, prev_hbm, ptr_hbm, out_hbm, ptr_v, buf0_v, buf1_v,
             outbuf_v, idx_v, sem0, sem1):
        wid = lax.axis_index("s") * _NC + lax.axis_index("c")
        pltpu.sync_copy(ptr_hbm, ptr_v)

        t_lo = row_lo + wid * rw
        t_hi = row_lo + (wid + 1) * rw
        f_lo = _count_below(ptr_v, t_lo)
        f_hi = (jnp.int32(_NSEG_PAD) if is_last
                else _count_below(ptr_v, t_hi))
        f_hi = jnp.where(wid == _NW - 1, f_hi, _count_below(ptr_v, t_hi))
        nsegs = f_hi - f_lo

        iota = lax.broadcasted_iota(jnp.int32, (_LANES,), 0)
        dump = _NSEG_PAD + wid

        def _flush(batch, count):
            base = f_lo + batch * _FB
            for half in range(_FB // _LANES):
                ids = base + half * _LANES + iota
                valid = (half * _LANES + iota) < count
                idx_v[pl.ds(half * _LANES, _LANES)] = jnp.where(
                    valid, ids, dump)
            pltpu.sync_copy(outbuf_v, out_hbm.at[idx_v])

        def _accum(buf, off, lo, hi, accs):
            rlo = jnp.maximum(lo - off, 0)
            rhi = jnp.minimum(hi - off, _CH)

            def row_body(r, accs):
                return tuple(accs[k] + buf[r, pl.ds(k * _LANES, _LANES)]
                             for k in range(8))

            return lax.fori_loop(rlo, rhi, row_body, accs)

        def _sum_range(src, lo, hi, accs):
            abase = (jnp.maximum(lo, 0) // 8) * 8
            nch = (hi - abase + _CH - 1) // _CH
            nch = jnp.where(hi > lo, nch, 0)

            def _fetch(off, buf, sem):
                pltpu.make_async_copy(src.at[pl.ds(off, _CH)], buf,
                                      sem).start()

            @pl.when(nch > 0)
            def _():
                _fetch(abase, buf0_v, sem0)

            def pair_body(cc, accs):
                c0 = 2 * cc
                off0 = abase + c0 * _CH
                pltpu.make_async_copy(src.at[pl.ds(off0, _CH)], buf0_v,
                                      sem0).wait()
                odd = c0 + 1 < nch

                @pl.when(odd)
                def _():
                    _fetch(off0 + _CH, buf1_v, sem1)

                accs = _accum(buf0_v, off0, lo, hi, accs)

                @pl.when(odd)
                def _():
                    pltpu.make_async_copy(src.at[pl.ds(off0 + _CH, _CH)],
                                          buf1_v, sem1).wait()

                @pl.when(c0 + 2 < nch)
                def _():
                    _fetch(off0 + 2 * _CH, buf0_v, sem0)

                return _accum(buf1_v, off0 + _CH, lo, hi, accs)

            return lax.fori_loop(0, (nch + 1) // 2, pair_body, accs)

        def seg_body(j, carry):
            seg = f_lo + j
            start = _extract(ptr_v, seg)
            end = _extract(ptr_v, seg + 1)
            accs = tuple(jnp.zeros((_LANES,), jnp.float32) for _ in range(8))
            if row_lo > 0:
                accs = _sum_range(prev_hbm, start,
                                  jnp.minimum(end, row_lo), accs)
            accs = _sum_range(h_hbm, jnp.maximum(start, row_lo) - row_lo,
                              end - row_lo, accs)
            slot = jnp.bitwise_and(j, _FB - 1)
            for k in range(8):
                outbuf_v[slot, pl.ds(k * _LANES, _LANES)] = accs[k]

            @pl.when(slot == _FB - 1)
            def _():
                _flush(j // _FB, _FB)

            return carry

        lax.fori_loop(0, nsegs, seg_body, 0)
        rem = jnp.bitwise_and(nsegs, _FB - 1)

        @pl.when(rem > 0)
        def _():
            _flush(nsegs // _FB, rem)

    return body


def kernel(h_node, x, ptr, W1, b1, W2, b2):
    n, embed_dim = h_node.shape
    nfeat = x.shape[1]
    nseg = ptr.shape[0] - 1
    hidden_dim = W1.shape[1]

    block_rows = 16000
    r_split = n // 2
    assert r_split % block_rows == 0 and r_split % (8 * _NW) == 0
    w1x = W1[:nfeat].astype(jnp.bfloat16)
    w1h = W1[nfeat:].astype(jnp.bfloat16)
    w2 = W2.astype(jnp.bfloat16)

    def mlp_half(row0):
        blk0 = row0 // block_rows
        nrows = r_split

        def in_map(i):
            return (i + blk0, 0)

        return pl.pallas_call(
            _tc_mlp_body,
            grid=(nrows // block_rows,),
            in_specs=[
                pl.BlockSpec((nfeat, hidden_dim), lambda i: (0, 0)),
                pl.BlockSpec((embed_dim, hidden_dim), lambda i: (0, 0)),
                pl.BlockSpec((hidden_dim, embed_dim), lambda i: (0, 0)),
                pl.BlockSpec((1, hidden_dim), lambda i: (0, 0)),
                pl.BlockSpec((1, embed_dim), lambda i: (0, 0)),
                pl.BlockSpec((block_rows, nfeat), in_map),
                pl.BlockSpec((block_rows, embed_dim), in_map),
            ],
            out_specs=pl.BlockSpec((block_rows, embed_dim), lambda i: (i, 0)),
            out_shape=jax.ShapeDtypeStruct((nrows + _CH, embed_dim),
                                           jnp.float32),
            compiler_params=pltpu.CompilerParams(
                dimension_semantics=("parallel",),
            ),
        )(w1x, w1h, w2, b1.reshape(1, -1), b2.reshape(1, -1), x, h_node)

    h0 = mlp_half(0)
    h1 = mlp_half(r_split)

    ptr32 = ptr.astype(jnp.int32)
    ptr_pad = jnp.concatenate(
        [ptr32, jnp.full((_PTR_BUF - (nseg + 1),), n, jnp.int32)])

    mesh = plsc.VectorSubcoreMesh(core_axis_name="c", subcore_axis_name="s",
                                  num_cores=_NC, num_subcores=_NS)
    scratch = [
        pltpu.VMEM((_PTR_BUF,), jnp.int32),
        pltpu.VMEM((_CH, embed_dim), jnp.float32),
        pltpu.VMEM((_CH, embed_dim), jnp.float32),
        pltpu.VMEM((_FB, embed_dim), jnp.float32),
        pltpu.VMEM((_FB,), jnp.int32),
        pltpu.SemaphoreType.DMA,
        pltpu.SemaphoreType.DMA,
    ]
    out_t = jax.ShapeDtypeStruct((_NSEG_PAD + _NW, embed_dim), jnp.float32)
    sc0 = pl.kernel(_make_sc_body(0, r_split, False), out_type=out_t,
                    mesh=mesh, scratch_types=scratch)(h0, h0, ptr_pad)
    sc1 = pl.kernel(_make_sc_body(r_split, n, True), out_type=out_t,
                    mesh=mesh, scratch_types=scratch)(h1, h0, ptr_pad)

    ends = jnp.concatenate(
        [ptr32[1:], jnp.full((_NSEG_PAD + _NW - nseg,), n, jnp.int32)])
    out = pl.pallas_call(
        functools.partial(_select_body, r_split=r_split),
        grid=(1,),
        in_specs=[
            pl.BlockSpec((_NSEG_PAD + _NW, 1), lambda i: (0, 0)),
            pl.BlockSpec((_NSEG_PAD + _NW, embed_dim), lambda i: (0, 0)),
            pl.BlockSpec((_NSEG_PAD + _NW, embed_dim), lambda i: (0, 0)),
        ],
        out_specs=pl.BlockSpec((_NSEG_PAD + _NW, embed_dim),
                               lambda i: (0, 0)),
        out_shape=jax.ShapeDtypeStruct((_NSEG_PAD + _NW, embed_dim),
                                       jnp.float32),
    )(ends.reshape(-1, 1), sc0, sc1)
    return out[:nseg]


# speedup vs baseline: 1.0996x; 1.0996x over previous
import functools

import jax
import jax.numpy as jnp
from jax import lax
from jax.experimental import pallas as pl
from jax.experimental.pallas import tpu as pltpu
from jax.experimental.pallas import tpu_sc as plsc

_LANES = 16
_CH = 256
_NC = 2
_NS = 16
_NW = _NC * _NS
_NSEG_PAD = 1024
_PTR_BUF = _NSEG_PAD + 2 * _LANES
_FB = 32


def _tc_mlp_body(w1x_ref, w1h_ref, w2_ref, b1_ref, b2_ref, x_ref, h_ref,
                 out_ref):
    xb = x_ref[...].astype(jnp.bfloat16)
    hb = h_ref[...].astype(jnp.bfloat16)
    hidden = jnp.maximum(
        jnp.dot(xb, w1x_ref[...], preferred_element_type=jnp.float32)
        + jnp.dot(hb, w1h_ref[...], preferred_element_type=jnp.float32)
        + b1_ref[...], 0.0)
    out_ref[...] = jnp.dot(hidden.astype(jnp.bfloat16), w2_ref[...],
                           preferred_element_type=jnp.float32) + b2_ref[...]


def _extract(vec_ref, j):
    v = vec_ref[pl.ds(j, _LANES)]
    return v[0]


def _count_below(ptr_v, thresh):
    base = jnp.int32(0)
    step = _NSEG_PAD
    while step >= 1:
        nxt = base + step
        safe = jnp.minimum(nxt, _PTR_BUF - _LANES)
        v = _extract(ptr_v, safe)
        take = jnp.logical_and(v < thresh, nxt <= _NSEG_PAD)
        base = jnp.where(take, nxt, base)
        step //= 2
    return base


def _sc_segsum_body(h_hbm, ptr_hbm, out_hbm, ptr_v, buf0_v, buf1_v, outbuf_v,
                    idx_v, sem0, sem1):
    wid = lax.axis_index("s") * _NC + lax.axis_index("c")
    n_rows = h_hbm.shape[0] - _CH
    rw = n_rows // _NW
    pltpu.sync_copy(ptr_hbm, ptr_v)

    t_lo = wid * rw
    t_hi = (wid + 1) * rw
    f_lo = _count_below(ptr_v, t_lo)
    f_hi = jnp.where(wid == _NW - 1, _NSEG_PAD, _count_below(ptr_v, t_hi))
    nsegs = f_hi - f_lo

    iota = lax.broadcasted_iota(jnp.int32, (_LANES,), 0)
    dump = _NSEG_PAD + wid

    def _flush(batch, count):
        base = f_lo + batch * _FB
        for half in range(_FB // _LANES):
            ids = base + half * _LANES + iota
            valid = (half * _LANES + iota) < count
            idx_v[pl.ds(half * _LANES, _LANES)] = jnp.where(valid, ids, dump)
        pltpu.sync_copy(outbuf_v, out_hbm.at[idx_v])

    def _fetch(off, buf, sem):
        pltpu.make_async_copy(h_hbm.at[pl.ds(off, _CH)], buf, sem).start()

    def _accum(buf, off, start, end, accs):
        rlo = jnp.maximum(start - off, 0)
        rhi = jnp.minimum(end - off, _CH)

        def row_body(r, accs):
            return tuple(accs[k] + buf[r, pl.ds(k * _LANES, _LANES)]
                         for k in range(8))

        return lax.fori_loop(rlo, rhi, row_body, accs)

    def seg_body(j, carry):
        seg = f_lo + j
        start = _extract(ptr_v, seg)
        end = _extract(ptr_v, seg + 1)
        abase = (start // 8) * 8
        nch = (end - abase + _CH - 1) // _CH

        @pl.when(nch > 0)
        def _():
            _fetch(abase, buf0_v, sem0)

        def pair_body(cc, accs):
            c0 = 2 * cc
            off0 = abase + c0 * _CH
            pltpu.make_async_copy(h_hbm.at[pl.ds(off0, _CH)], buf0_v,
                                  sem0).wait()
            odd = c0 + 1 < nch

            @pl.when(odd)
            def _():
                _fetch(off0 + _CH, buf1_v, sem1)

            accs = _accum(buf0_v, off0, start, end, accs)

            @pl.when(odd)
            def _():
                pltpu.make_async_copy(h_hbm.at[pl.ds(off0 + _CH, _CH)],
                                      buf1_v, sem1).wait()

            @pl.when(c0 + 2 < nch)
            def _():
                _fetch(off0 + 2 * _CH, buf0_v, sem0)

            return _accum(buf1_v, off0 + _CH, start, end, accs)

        zeros = tuple(jnp.zeros((_LANES,), jnp.float32) for _ in range(8))
        accs = lax.fori_loop(0, (nch + 1) // 2, pair_body, zeros)
        slot = jnp.bitwise_and(j, _FB - 1)
        for k in range(8):
            outbuf_v[slot, pl.ds(k * _LANES, _LANES)] = accs[k]

        @pl.when(slot == _FB - 1)
        def _():
            _flush(j // _FB, _FB)

        return carry

    lax.fori_loop(0, nsegs, seg_body, 0)
    rem = jnp.bitwise_and(nsegs, _FB - 1)

    @pl.when(rem > 0)
    def _():
        _flush(nsegs // _FB, rem)


def kernel(h_node, x, ptr, W1, b1, W2, b2):
    n, embed_dim = h_node.shape
    nfeat = x.shape[1]
    nseg = ptr.shape[0] - 1
    hidden_dim = W1.shape[1]

    block_rows = 16000
    grid = (n // block_rows,)
    w1x = W1[:nfeat].astype(jnp.bfloat16)
    w1h = W1[nfeat:].astype(jnp.bfloat16)
    w2 = W2.astype(jnp.bfloat16)

    h = pl.pallas_call(
        _tc_mlp_body,
        grid=grid,
        in_specs=[
            pl.BlockSpec((nfeat, hidden_dim), lambda i: (0, 0)),
            pl.BlockSpec((embed_dim, hidden_dim), lambda i: (0, 0)),
            pl.BlockSpec((hidden_dim, embed_dim), lambda i: (0, 0)),
            pl.BlockSpec((1, hidden_dim), lambda i: (0, 0)),
            pl.BlockSpec((1, embed_dim), lambda i: (0, 0)),
            pl.BlockSpec((block_rows, nfeat), lambda i: (i, 0)),
            pl.BlockSpec((block_rows, embed_dim), lambda i: (i, 0)),
        ],
        out_specs=pl.BlockSpec((block_rows, embed_dim), lambda i: (i, 0)),
        out_shape=jax.ShapeDtypeStruct((n + _CH, embed_dim), jnp.float32),
        compiler_params=pltpu.CompilerParams(
            dimension_semantics=("parallel",),
        ),
    )(w1x, w1h, w2, b1.reshape(1, -1), b2.reshape(1, -1), x, h_node)

    ptr32 = ptr.astype(jnp.int32)
    ptr_pad = jnp.concatenate(
        [ptr32, jnp.full((_PTR_BUF - (nseg + 1),), n, jnp.int32)])

    mesh = plsc.VectorSubcoreMesh(core_axis_name="c", subcore_axis_name="s",
                                  num_cores=_NC, num_subcores=_NS)
    sc_out = pl.kernel(
        _sc_segsum_body,
        out_type=jax.ShapeDtypeStruct((_NSEG_PAD + _NW, embed_dim),
                                      jnp.float32),
        mesh=mesh,
        scratch_types=[
            pltpu.VMEM((_PTR_BUF,), jnp.int32),
            pltpu.VMEM((_CH, embed_dim), jnp.float32),
            pltpu.VMEM((_CH, embed_dim), jnp.float32),
            pltpu.VMEM((_FB, embed_dim), jnp.float32),
            pltpu.VMEM((_FB,), jnp.int32),
            pltpu.SemaphoreType.DMA,
            pltpu.SemaphoreType.DMA,
        ],
    )(h, ptr_pad)
    return sc_out[:nseg]


# speedup vs baseline: 1.3323x; 1.2116x over previous
import functools

import jax
import jax.numpy as jnp
from jax import lax
from jax.experimental import pallas as pl
from jax.experimental.pallas import tpu as pltpu
from jax.experimental.pallas import tpu_sc as plsc

_LANES = 16
_CH = 256
_NC = 2
_NS = 16
_NW = _NC * _NS
_NSEG_PAD = 1024
_PTR_BUF = _NSEG_PAD + 2 * _LANES
_FB = 32


def _tc_mlp_body(w1x_ref, w1h_ref, w2_ref, b1_ref, b2_ref, x_ref, h_ref,
                 out_ref):
    xb = x_ref[...].astype(jnp.bfloat16)
    hb = h_ref[...].astype(jnp.bfloat16)
    hidden = jnp.maximum(
        jnp.dot(xb, w1x_ref[...], preferred_element_type=jnp.float32)
        + jnp.dot(hb, w1h_ref[...], preferred_element_type=jnp.float32)
        + b1_ref[...], 0.0)
    out_ref[...] = jnp.dot(hidden.astype(jnp.bfloat16), w2_ref[...],
                           preferred_element_type=jnp.float32) + b2_ref[...]


def _extract(vec_ref, j):
    v = vec_ref[pl.ds(j, _LANES)]
    return v[0]


def _count_below(ptr_v, thresh):
    base = jnp.int32(0)
    step = _NSEG_PAD
    while step >= 1:
        nxt = base + step
        safe = jnp.minimum(nxt, _PTR_BUF - _LANES)
        v = _extract(ptr_v, safe)
        take = jnp.logical_and(v < thresh, nxt <= _NSEG_PAD)
        base = jnp.where(take, nxt, base)
        step //= 2
    return base


def _sc_segsum_body(h_hbm, ptr_hbm, out_hbm, ptr_v, buf0_v, buf1_v, outbuf_v,
                    idx_v, sem0, sem1):
    wid = lax.axis_index("s") * _NC + lax.axis_index("c")
    n_rows = h_hbm.shape[0] - _CH
    rw = n_rows // _NW
    pltpu.sync_copy(ptr_hbm, ptr_v)

    t_lo = wid * rw
    t_hi = (wid + 1) * rw
    f_lo = _count_below(ptr_v, t_lo)
    f_hi = jnp.where(wid == _NW - 1, _NSEG_PAD, _count_below(ptr_v, t_hi))
    nsegs = f_hi - f_lo

    iota = lax.broadcasted_iota(jnp.int32, (_LANES,), 0)
    dump = _NSEG_PAD + wid

    def _flush(batch, count):
        base = f_lo + batch * _FB
        for half in range(_FB // _LANES):
            ids = base + half * _LANES + iota
            valid = (half * _LANES + iota) < count
            idx_v[pl.ds(half * _LANES, _LANES)] = jnp.where(valid, ids, dump)
        pltpu.sync_copy(outbuf_v, out_hbm.at[idx_v])

    def _fetch(off, buf, sem):
        pltpu.make_async_copy(h_hbm.at[pl.ds(off, _CH)], buf, sem).start()

    def _accum_range(buf, off, lo, hi, accs):
        rlo = jnp.maximum(lo - off, 0)
        rhi = jnp.minimum(hi - off, _CH)

        def row_body(r, accs):
            return tuple(accs[k] + buf[r, pl.ds(k * _LANES, _LANES)]
                         for k in range(8))

        return lax.fori_loop(rlo, rhi, row_body, accs)

    wstart = _extract(ptr_v, f_lo)
    wend = _extract(ptr_v, jnp.minimum(f_hi, _NSEG_PAD))
    abase = (wstart // 8) * 8
    nch = (wend - abase + _CH - 1) // _CH
    nch = jnp.where(nsegs > 0, nch, 0)

    zeros = tuple(jnp.zeros((_LANES,), jnp.float32) for _ in range(8))

    def _chunk(buf, off, carry):
        q, accs = carry[0], carry[1:]
        chunk_hi = off + _CH
        q_hi = jnp.minimum(_count_below(ptr_v, chunk_hi + 1), f_hi)

        def drain(qq, accs):
            start = _extract(ptr_v, qq)
            end = _extract(ptr_v, qq + 1)
            accs = _accum_range(buf, off, jnp.maximum(start, off), end, accs)
            j = qq - f_lo
            slot = jnp.bitwise_and(j, _FB - 1)
            for k in range(8):
                outbuf_v[slot, pl.ds(k * _LANES, _LANES)] = accs[k]

            @pl.when(slot == _FB - 1)
            def _():
                _flush(j // _FB, _FB)

            return zeros

        accs = lax.fori_loop(q, q_hi, drain, accs)
        start = _extract(ptr_v, jnp.minimum(q_hi, _NSEG_PAD))
        hi = jnp.where(q_hi < f_hi, chunk_hi, off)
        accs = _accum_range(buf, off, jnp.maximum(start, off), hi, accs)
        return (q_hi,) + accs

    @pl.when(nch > 0)
    def _():
        _fetch(abase, buf0_v, sem0)

    def pair_body(cc, carry):
        c0 = 2 * cc
        off0 = abase + c0 * _CH
        pltpu.make_async_copy(h_hbm.at[pl.ds(off0, _CH)], buf0_v,
                              sem0).wait()
        odd = c0 + 1 < nch

        @pl.when(odd)
        def _():
            _fetch(off0 + _CH, buf1_v, sem1)

        carry = _chunk(buf0_v, off0, carry)

        @pl.when(odd)
        def _():
            pltpu.make_async_copy(h_hbm.at[pl.ds(off0 + _CH, _CH)],
                                  buf1_v, sem1).wait()

        @pl.when(c0 + 2 < nch)
        def _():
            _fetch(off0 + 2 * _CH, buf0_v, sem0)

        return _chunk(buf1_v, off0 + _CH, carry)

    lax.fori_loop(0, (nch + 1) // 2, pair_body, (f_lo,) + zeros)
    rem = jnp.bitwise_and(nsegs, _FB - 1)

    @pl.when(rem > 0)
    def _():
        _flush(nsegs // _FB, rem)


def kernel(h_node, x, ptr, W1, b1, W2, b2):
    n, embed_dim = h_node.shape
    nfeat = x.shape[1]
    nseg = ptr.shape[0] - 1
    hidden_dim = W1.shape[1]

    block_rows = 16000
    grid = (n // block_rows,)
    w1x = W1[:nfeat].astype(jnp.bfloat16)
    w1h = W1[nfeat:].astype(jnp.bfloat16)
    w2 = W2.astype(jnp.bfloat16)

    h = pl.pallas_call(
        _tc_mlp_body,
        grid=grid,
        in_specs=[
            pl.BlockSpec((nfeat, hidden_dim), lambda i: (0, 0)),
            pl.BlockSpec((embed_dim, hidden_dim), lambda i: (0, 0)),
            pl.BlockSpec((hidden_dim, embed_dim), lambda i: (0, 0)),
            pl.BlockSpec((1, hidden_dim), lambda i: (0, 0)),
            pl.BlockSpec((1, embed_dim), lambda i: (0, 0)),
            pl.BlockSpec((block_rows, nfeat), lambda i: (i, 0)),
            pl.BlockSpec((block_rows, embed_dim), lambda i: (i, 0)),
        ],
        out_specs=pl.BlockSpec((block_rows, embed_dim), lambda i: (i, 0)),
        out_shape=jax.ShapeDtypeStruct((n + _CH, embed_dim), jnp.float32),
        compiler_params=pltpu.CompilerParams(
            dimension_semantics=("parallel",),
        ),
    )(w1x, w1h, w2, b1.reshape(1, -1), b2.reshape(1, -1), x, h_node)

    ptr32 = ptr.astype(jnp.int32)
    ptr_pad = jnp.concatenate(
        [ptr32, jnp.full((_PTR_BUF - (nseg + 1),), n, jnp.int32)])

    mesh = plsc.VectorSubcoreMesh(core_axis_name="c", subcore_axis_name="s",
                                  num_cores=_NC, num_subcores=_NS)
    sc_out = pl.kernel(
        _sc_segsum_body,
        out_type=jax.ShapeDtypeStruct((_NSEG_PAD + _NW, embed_dim),
                                      jnp.float32),
        mesh=mesh,
        scratch_types=[
            pltpu.VMEM((_PTR_BUF,), jnp.int32),
            pltpu.VMEM((_CH, embed_dim), jnp.float32),
            pltpu.VMEM((_CH, embed_dim), jnp.float32),
            pltpu.VMEM((_FB, embed_dim), jnp.float32),
            pltpu.VMEM((_FB,), jnp.int32),
            pltpu.SemaphoreType.DMA,
            pltpu.SemaphoreType.DMA,
        ],
    )(h, ptr_pad)
    return sc_out[:nseg]


# speedup vs baseline: 1.3780x; 1.0343x over previous
import functools

import jax
import jax.numpy as jnp
from jax import lax
from jax.experimental import pallas as pl
from jax.experimental.pallas import tpu as pltpu
from jax.experimental.pallas import tpu_sc as plsc

_LANES = 16
_CH = 384
_NC = 2
_NS = 16
_NW = _NC * _NS
_NSEG_PAD = 1024
_PTR_BUF = _NSEG_PAD + 2 * _LANES
_FB = 32


def _tc_mlp_body(w1x_ref, w1h_ref, w2_ref, b1_ref, b2_ref, x_ref, h_ref,
                 out_ref):
    xb = x_ref[...].astype(jnp.bfloat16)
    hb = h_ref[...].astype(jnp.bfloat16)
    hidden = jnp.maximum(
        jnp.dot(xb, w1x_ref[...], preferred_element_type=jnp.float32)
        + jnp.dot(hb, w1h_ref[...], preferred_element_type=jnp.float32)
        + b1_ref[...], 0.0)
    out_ref[...] = jnp.dot(hidden.astype(jnp.bfloat16), w2_ref[...],
                           preferred_element_type=jnp.float32) + b2_ref[...]


def _extract(vec_ref, j):
    v = vec_ref[pl.ds(j, _LANES)]
    return v[0]


def _count_below(ptr_v, thresh):
    base = jnp.int32(0)
    step = _NSEG_PAD
    while step >= 1:
        nxt = base + step
        safe = jnp.minimum(nxt, _PTR_BUF - _LANES)
        v = _extract(ptr_v, safe)
        take = jnp.logical_and(v < thresh, nxt <= _NSEG_PAD)
        base = jnp.where(take, nxt, base)
        step //= 2
    return base


def _sc_segsum_body(h_hbm, ptr_hbm, out_hbm, ptr_v, buf0_v, buf1_v, outbuf_v,
                    idx_v, sem0, sem1):
    wid = lax.axis_index("s") * _NC + lax.axis_index("c")
    n_rows = h_hbm.shape[0] - _CH
    rw = n_rows // _NW
    pltpu.sync_copy(ptr_hbm, ptr_v)

    t_lo = wid * rw
    t_hi = (wid + 1) * rw
    f_lo = _count_below(ptr_v, t_lo)
    f_hi = jnp.where(wid == _NW - 1, _NSEG_PAD, _count_below(ptr_v, t_hi))
    nsegs = f_hi - f_lo

    iota = lax.broadcasted_iota(jnp.int32, (_LANES,), 0)
    dump = _NSEG_PAD + wid

    def _flush(batch, count):
        base = f_lo + batch * _FB
        for half in range(_FB // _LANES):
            ids = base + half * _LANES + iota
            valid = (half * _LANES + iota) < count
            idx_v[pl.ds(half * _LANES, _LANES)] = jnp.where(valid, ids, dump)
        pltpu.sync_copy(outbuf_v, out_hbm.at[idx_v])

    def _fetch(off, buf, sem):
        pltpu.make_async_copy(h_hbm.at[pl.ds(off, _CH)], buf, sem).start()

    def _accum_range(buf, off, lo, hi, accs):
        rlo = jnp.maximum(lo - off, 0)
        rhi = jnp.minimum(hi - off, _CH)

        def row_body(r, accs):
            return tuple(accs[k] + buf[r, pl.ds(k * _LANES, _LANES)]
                         for k in range(8))

        return lax.fori_loop(rlo, rhi, row_body, accs)

    wstart = _extract(ptr_v, f_lo)
    wend = _extract(ptr_v, jnp.minimum(f_hi, _NSEG_PAD))
    abase = (wstart // 8) * 8
    nch = (wend - abase + _CH - 1) // _CH
    nch = jnp.where(nsegs > 0, nch, 0)

    zeros = tuple(jnp.zeros((_LANES,), jnp.float32) for _ in range(8))

    def _chunk(buf, off, carry):
        q, accs = carry[0], carry[1:]
        chunk_hi = off + _CH
        q_hi = jnp.minimum(_count_below(ptr_v, chunk_hi + 1), f_hi)

        def drain(qq, accs):
            start = _extract(ptr_v, qq)
            end = _extract(ptr_v, qq + 1)
            accs = _accum_range(buf, off, jnp.maximum(start, off), end, accs)
            j = qq - f_lo
            slot = jnp.bitwise_and(j, _FB - 1)
            for k in range(8):
                outbuf_v[slot, pl.ds(k * _LANES, _LANES)] = accs[k]

            @pl.when(slot == _FB - 1)
            def _():
                _flush(j // _FB, _FB)

            return zeros

        accs = lax.fori_loop(q, q_hi, drain, accs)
        start = _extract(ptr_v, jnp.minimum(q_hi, _NSEG_PAD))
        hi = jnp.where(q_hi < f_hi, chunk_hi, off)
        accs = _accum_range(buf, off, jnp.maximum(start, off), hi, accs)
        return (q_hi,) + accs

    @pl.when(nch > 0)
    def _():
        _fetch(abase, buf0_v, sem0)

    def pair_body(cc, carry):
        c0 = 2 * cc
        off0 = abase + c0 * _CH
        pltpu.make_async_copy(h_hbm.at[pl.ds(off0, _CH)], buf0_v,
                              sem0).wait()
        odd = c0 + 1 < nch

        @pl.when(odd)
        def _():
            _fetch(off0 + _CH, buf1_v, sem1)

        carry = _chunk(buf0_v, off0, carry)

        @pl.when(odd)
        def _():
            pltpu.make_async_copy(h_hbm.at[pl.ds(off0 + _CH, _CH)],
                                  buf1_v, sem1).wait()

        @pl.when(c0 + 2 < nch)
        def _():
            _fetch(off0 + 2 * _CH, buf0_v, sem0)

        return _chunk(buf1_v, off0 + _CH, carry)

    lax.fori_loop(0, (nch + 1) // 2, pair_body, (f_lo,) + zeros)
    rem = jnp.bitwise_and(nsegs, _FB - 1)

    @pl.when(rem > 0)
    def _():
        _flush(nsegs // _FB, rem)


def kernel(h_node, x, ptr, W1, b1, W2, b2):
    n, embed_dim = h_node.shape
    nfeat = x.shape[1]
    nseg = ptr.shape[0] - 1
    hidden_dim = W1.shape[1]

    block_rows = 16000
    grid = (n // block_rows,)
    w1x = W1[:nfeat].astype(jnp.bfloat16)
    w1h = W1[nfeat:].astype(jnp.bfloat16)
    w2 = W2.astype(jnp.bfloat16)

    h = pl.pallas_call(
        _tc_mlp_body,
        grid=grid,
        in_specs=[
            pl.BlockSpec((nfeat, hidden_dim), lambda i: (0, 0)),
            pl.BlockSpec((embed_dim, hidden_dim), lambda i: (0, 0)),
            pl.BlockSpec((hidden_dim, embed_dim), lambda i: (0, 0)),
            pl.BlockSpec((1, hidden_dim), lambda i: (0, 0)),
            pl.BlockSpec((1, embed_dim), lambda i: (0, 0)),
            pl.BlockSpec((block_rows, nfeat), lambda i: (i, 0)),
            pl.BlockSpec((block_rows, embed_dim), lambda i: (i, 0)),
        ],
        out_specs=pl.BlockSpec((block_rows, embed_dim), lambda i: (i, 0)),
        out_shape=jax.ShapeDtypeStruct((n + _CH, embed_dim), jnp.float32),
        compiler_params=pltpu.CompilerParams(
            dimension_semantics=("parallel",),
        ),
    )(w1x, w1h, w2, b1.reshape(1, -1), b2.reshape(1, -1), x, h_node)

    ptr32 = ptr.astype(jnp.int32)
    ptr_pad = jnp.concatenate(
        [ptr32, jnp.full((_PTR_BUF - (nseg + 1),), n, jnp.int32)])

    mesh = plsc.VectorSubcoreMesh(core_axis_name="c", subcore_axis_name="s",
                                  num_cores=_NC, num_subcores=_NS)
    sc_out = pl.kernel(
        _sc_segsum_body,
        out_type=jax.ShapeDtypeStruct((_NSEG_PAD + _NW, embed_dim),
                                      jnp.float32),
        mesh=mesh,
        scratch_types=[
            pltpu.VMEM((_PTR_BUF,), jnp.int32),
            pltpu.VMEM((_CH, embed_dim), jnp.float32),
            pltpu.VMEM((_CH, embed_dim), jnp.float32),
            pltpu.VMEM((_FB, embed_dim), jnp.float32),
            pltpu.VMEM((_FB,), jnp.int32),
            pltpu.SemaphoreType.DMA,
            pltpu.SemaphoreType.DMA,
        ],
    )(h, ptr_pad)
    return sc_out[:nseg]


# speedup vs baseline: 1.3902x; 1.0088x over previous
import functools

import jax
import jax.numpy as jnp
from jax import lax
from jax.experimental import pallas as pl
from jax.experimental.pallas import tpu as pltpu
from jax.experimental.pallas import tpu_sc as plsc

_LANES = 16
_CH = 448
_NC = 2
_NS = 16
_NW = _NC * _NS
_NSEG_PAD = 1024
_PTR_BUF = _NSEG_PAD + 2 * _LANES
_FB = 32


def _tc_mlp_body(w1x_ref, w1h_ref, w2_ref, b1_ref, b2_ref, x_ref, h_ref,
                 out_ref):
    xb = x_ref[...].astype(jnp.bfloat16)
    hb = h_ref[...].astype(jnp.bfloat16)
    hidden = jnp.maximum(
        jnp.dot(xb, w1x_ref[...], preferred_element_type=jnp.float32)
        + jnp.dot(hb, w1h_ref[...], preferred_element_type=jnp.float32)
        + b1_ref[...], 0.0)
    out_ref[...] = jnp.dot(hidden.astype(jnp.bfloat16), w2_ref[...],
                           preferred_element_type=jnp.float32) + b2_ref[...]


def _extract(vec_ref, j):
    v = vec_ref[pl.ds(j, _LANES)]
    return v[0]


def _count_below(ptr_v, thresh):
    base = jnp.int32(0)
    step = _NSEG_PAD
    while step >= 1:
        nxt = base + step
        safe = jnp.minimum(nxt, _PTR_BUF - _LANES)
        v = _extract(ptr_v, safe)
        take = jnp.logical_and(v < thresh, nxt <= _NSEG_PAD)
        base = jnp.where(take, nxt, base)
        step //= 2
    return base


def _sc_segsum_body(h_hbm, ptr_hbm, out_hbm, ptr_v, buf0_v, buf1_v, outbuf_v,
                    idx_v, sem0, sem1):
    wid = lax.axis_index("s") * _NC + lax.axis_index("c")
    n_rows = h_hbm.shape[0] - _CH
    rw = n_rows // _NW
    pltpu.sync_copy(ptr_hbm, ptr_v)

    t_lo = wid * rw
    t_hi = (wid + 1) * rw
    f_lo = _count_below(ptr_v, t_lo)
    f_hi = jnp.where(wid == _NW - 1, _NSEG_PAD, _count_below(ptr_v, t_hi))
    nsegs = f_hi - f_lo

    iota = lax.broadcasted_iota(jnp.int32, (_LANES,), 0)
    dump = _NSEG_PAD + wid

    def _flush(batch, count):
        base = f_lo + batch * _FB
        for half in range(_FB // _LANES):
            ids = base + half * _LANES + iota
            valid = (half * _LANES + iota) < count
            idx_v[pl.ds(half * _LANES, _LANES)] = jnp.where(valid, ids, dump)
        pltpu.sync_copy(outbuf_v, out_hbm.at[idx_v])

    def _fetch(off, buf, sem):
        pltpu.make_async_copy(h_hbm.at[pl.ds(off, _CH)], buf, sem).start()

    def _accum_range(buf, off, lo, hi, accs):
        rlo = jnp.maximum(lo - off, 0)
        rhi = jnp.minimum(hi - off, _CH)

        def row_body(r, accs):
            return tuple(accs[k] + buf[r, pl.ds(k * _LANES, _LANES)]
                         for k in range(8))

        return lax.fori_loop(rlo, rhi, row_body, accs)

    wstart = _extract(ptr_v, f_lo)
    wend = _extract(ptr_v, jnp.minimum(f_hi, _NSEG_PAD))
    abase = (wstart // 8) * 8
    nch = (wend - abase + _CH - 1) // _CH
    nch = jnp.where(nsegs > 0, nch, 0)

    zeros = tuple(jnp.zeros((_LANES,), jnp.float32) for _ in range(8))

    def _chunk(buf, off, carry):
        q, accs = carry[0], carry[1:]
        chunk_hi = off + _CH
        q_hi = jnp.minimum(_count_below(ptr_v, chunk_hi + 1), f_hi)

        def drain(qq, accs):
            start = _extract(ptr_v, qq)
            end = _extract(ptr_v, qq + 1)
            accs = _accum_range(buf, off, jnp.maximum(start, off), end, accs)
            j = qq - f_lo
            slot = jnp.bitwise_and(j, _FB - 1)
            for k in range(8):
                outbuf_v[slot, pl.ds(k * _LANES, _LANES)] = accs[k]

            @pl.when(slot == _FB - 1)
            def _():
                _flush(j // _FB, _FB)

            return zeros

        accs = lax.fori_loop(q, q_hi, drain, accs)
        start = _extract(ptr_v, jnp.minimum(q_hi, _NSEG_PAD))
        hi = jnp.where(q_hi < f_hi, chunk_hi, off)
        accs = _accum_range(buf, off, jnp.maximum(start, off), hi, accs)
        return (q_hi,) + accs

    @pl.when(nch > 0)
    def _():
        _fetch(abase, buf0_v, sem0)

    def pair_body(cc, carry):
        c0 = 2 * cc
        off0 = abase + c0 * _CH
        pltpu.make_async_copy(h_hbm.at[pl.ds(off0, _CH)], buf0_v,
                              sem0).wait()
        odd = c0 + 1 < nch

        @pl.when(odd)
        def _():
            _fetch(off0 + _CH, buf1_v, sem1)

        carry = _chunk(buf0_v, off0, carry)

        @pl.when(odd)
        def _():
            pltpu.make_async_copy(h_hbm.at[pl.ds(off0 + _CH, _CH)],
                                  buf1_v, sem1).wait()

        @pl.when(c0 + 2 < nch)
        def _():
            _fetch(off0 + 2 * _CH, buf0_v, sem0)

        return _chunk(buf1_v, off0 + _CH, carry)

    lax.fori_loop(0, (nch + 1) // 2, pair_body, (f_lo,) + zeros)
    rem = jnp.bitwise_and(nsegs, _FB - 1)

    @pl.when(rem > 0)
    def _():
        _flush(nsegs // _FB, rem)


def kernel(h_node, x, ptr, W1, b1, W2, b2):
    n, embed_dim = h_node.shape
    nfeat = x.shape[1]
    nseg = ptr.shape[0] - 1
    hidden_dim = W1.shape[1]

    block_rows = 16000
    grid = (n // block_rows,)
    w1x = W1[:nfeat].astype(jnp.bfloat16)
    w1h = W1[nfeat:].astype(jnp.bfloat16)
    w2 = W2.astype(jnp.bfloat16)

    h = pl.pallas_call(
        _tc_mlp_body,
        grid=grid,
        in_specs=[
            pl.BlockSpec((nfeat, hidden_dim), lambda i: (0, 0)),
            pl.BlockSpec((embed_dim, hidden_dim), lambda i: (0, 0)),
            pl.BlockSpec((hidden_dim, embed_dim), lambda i: (0, 0)),
            pl.BlockSpec((1, hidden_dim), lambda i: (0, 0)),
            pl.BlockSpec((1, embed_dim), lambda i: (0, 0)),
            pl.BlockSpec((block_rows, nfeat), lambda i: (i, 0)),
            pl.BlockSpec((block_rows, embed_dim), lambda i: (i, 0)),
        ],
        out_specs=pl.BlockSpec((block_rows, embed_dim), lambda i: (i, 0)),
        out_shape=jax.ShapeDtypeStruct((n + _CH, embed_dim), jnp.float32),
        compiler_params=pltpu.CompilerParams(
            dimension_semantics=("parallel",),
        ),
    )(w1x, w1h, w2, b1.reshape(1, -1), b2.reshape(1, -1), x, h_node)

    ptr32 = ptr.astype(jnp.int32)
    ptr_pad = jnp.concatenate(
        [ptr32, jnp.full((_PTR_BUF - (nseg + 1),), n, jnp.int32)])

    mesh = plsc.VectorSubcoreMesh(core_axis_name="c", subcore_axis_name="s",
                                  num_cores=_NC, num_subcores=_NS)
    sc_out = pl.kernel(
        _sc_segsum_body,
        out_type=jax.ShapeDtypeStruct((_NSEG_PAD + _NW, embed_dim),
                                      jnp.float32),
        mesh=mesh,
        scratch_types=[
            pltpu.VMEM((_PTR_BUF,), jnp.int32),
            pltpu.VMEM((_CH, embed_dim), jnp.float32),
            pltpu.VMEM((_CH, embed_dim), jnp.float32),
            pltpu.VMEM((_FB, embed_dim), jnp.float32),
            pltpu.VMEM((_FB,), jnp.int32),
            pltpu.SemaphoreType.DMA,
            pltpu.SemaphoreType.DMA,
        ],
    )(h, ptr_pad)
    return sc_out[:nseg]
